# Initial kernel scaffold; baseline (speedup 1.0000x reference)
import jax
import jax.numpy as jnp
from jax.experimental import pallas as pl


def kernel(h, coord, edge_attr, W_e1, b_e1, W_e2, b_e2, W_n1, b_n1, W_n2, b_n2, W_c1, b_c1, W_c2, edge_index):
    raise NotImplementedError("write your pallas kernel here")


# speedup vs baseline: 4.7832x; 4.7832x over previous
import functools

import jax
import jax.numpy as jnp
from jax import lax
from jax.experimental import pallas as pl
from jax.experimental.pallas import tpu as pltpu
from jax.experimental.pallas import tpu_sc as plsc

N = 10000
E = 320000
D = 128
D_EDGE = 4
HID = 128

NC = 2
NS = 16
NW = NC * NS
CHK = 128
QK = 32
NCHT = E // CHK
NFULL = NCHT // NW
NTAIL = NCHT - NFULL * NW
L = 16

NACC = 10240
RPT = NACC // NS

_mesh = plsc.VectorSubcoreMesh(core_axis_name="c", subcore_axis_name="s")


@functools.partial(
    pl.kernel,
    mesh=_mesh,
    compiler_params=pltpu.CompilerParams(needs_layout_passes=False),
    out_type=[
        jax.ShapeDtypeStruct((E, D), jnp.float32),
        jax.ShapeDtypeStruct((E, D), jnp.float32),
        jax.ShapeDtypeStruct((4, E), jnp.float32),
    ],
    scratch_types=[
        pltpu.VMEM((CHK,), jnp.int32),
        pltpu.VMEM((CHK,), jnp.int32),
        pltpu.VMEM((CHK, D), jnp.float32),
        pltpu.VMEM((CHK, D), jnp.float32),
        pltpu.VMEM((4, CHK), jnp.float32),
        pltpu.VMEM((3 * N,), jnp.float32),
        pltpu.SemaphoreType.DMA,
        pltpu.SemaphoreType.DMA,
    ],
)
def _gather(h_hbm, coordf_hbm, row_hbm, col_hbm, outr, outc, outd,
            idx_r, idx_c, rows_r, rows_c, dr_v, coord_v, sem1, sem2):
    wid = lax.axis_index("s") * NC + lax.axis_index("c")
    pltpu.sync_copy(coordf_hbm, coord_v)

    def chunk(ck):
        base = ck * CHK
        pltpu.sync_copy(row_hbm.at[pl.ds(base, CHK)], idx_r)
        pltpu.sync_copy(col_hbm.at[pl.ds(base, CHK)], idx_c)
        g1 = pltpu.async_copy(h_hbm.at[idx_r], rows_r, sem1)
        g2 = pltpu.async_copy(h_hbm.at[idx_c], rows_c, sem2)
        for g in range(CHK // L):
            ri3 = idx_r[pl.ds(g * L, L)] * 3
            ci3 = idx_c[pl.ds(g * L, L)] * 3
            dx = (plsc.load_gather(coord_v, [ri3])
                  - plsc.load_gather(coord_v, [ci3]))
            dy = (plsc.load_gather(coord_v, [ri3 + 1])
                  - plsc.load_gather(coord_v, [ci3 + 1]))
            dz = (plsc.load_gather(coord_v, [ri3 + 2])
                  - plsc.load_gather(coord_v, [ci3 + 2]))
            dr_v[0, pl.ds(g * L, L)] = dx
            dr_v[1, pl.ds(g * L, L)] = dy
            dr_v[2, pl.ds(g * L, L)] = dz
            dr_v[3, pl.ds(g * L, L)] = dx * dx + dy * dy + dz * dz
        pltpu.sync_copy(dr_v, outd.at[:, pl.ds(base, CHK)])
        g1.wait()
        pltpu.sync_copy(rows_r, outr.at[pl.ds(base, CHK)])
        g2.wait()
        pltpu.sync_copy(rows_c, outc.at[pl.ds(base, CHK)])

    def body(j, carry):
        chunk(j * NW + wid)
        return carry

    lax.fori_loop(0, NFULL, body, 0)

    @pl.when(wid < NTAIL)
    def _():
        chunk(NFULL * NW + wid)


@functools.partial(
    pl.kernel,
    mesh=_mesh,
    compiler_params=pltpu.CompilerParams(needs_layout_passes=False),
    out_type=[
        jax.ShapeDtypeStruct((NC, NACC, D), jnp.float32),
        jax.ShapeDtypeStruct((NW, 4 * N), jnp.float32),
    ],
    scratch_types=[
        [pltpu.VMEM((QK,), jnp.int32) for _ in range(CHK // QK)],
        pltpu.VMEM((QK, D), jnp.float32),
        pltpu.VMEM((4, CHK), jnp.float32),
        pltpu.VMEM((4 * N,), jnp.float32),
        pltpu.VMEM_SHARED((NACC, D), jnp.float32),
        pltpu.SemaphoreType.DMA,
    ],
)
def _scatter(ef_hbm, tdp_hbm, row_hbm, z2_hbm, zf_hbm, outh, out4,
             idx_q, data_v, td_v, acc4_v, acch_sh, sem):
    cid = lax.axis_index("c")
    sid = lax.axis_index("s")
    wid = sid * NC + cid

    pltpu.sync_copy(z2_hbm, data_v)
    for k in range(RPT // QK):
        pltpu.sync_copy(data_v, acch_sh.at[pl.ds(sid * RPT + k * QK, QK)])
    pltpu.sync_copy(zf_hbm, acc4_v)
    plsc.subcore_barrier()

    def chunk(ck):
        base = ck * CHK
        for q in range(CHK // QK):
            pltpu.sync_copy(row_hbm.at[pl.ds(base + q * QK, QK)], idx_q[q])
        pltpu.sync_copy(tdp_hbm.at[:, pl.ds(base, CHK)], td_v)
        for q in range(CHK // QK):
            pltpu.sync_copy(ef_hbm.at[pl.ds(base + q * QK, QK)], data_v)
            pltpu.sync_copy(data_v, acch_sh.at[idx_q[q]], add=True)
        for k in range(CHK // L):
            ids = idx_q[k // 2][pl.ds((k % 2) * L, L)]
            for p in range(4):
                val = td_v[p, pl.ds(k * L, L)]
                plsc.addupdate_scatter(acc4_v, [ids + p * N], val)

    def body(j, carry):
        chunk(j * NW + wid)
        return carry

    lax.fori_loop(0, NFULL, body, 0)

    @pl.when(wid < NTAIL)
    def _():
        chunk(NFULL * NW + wid)

    pltpu.sync_copy(acc4_v, out4.at[wid])
    plsc.subcore_barrier()

    for k in range(RPT // QK):
        off = sid * RPT + k * QK
        pltpu.sync_copy(acch_sh.at[pl.ds(off, QK)], data_v)
        pltpu.sync_copy(data_v, outh.at[cid, pl.ds(off, QK)])


def _silu(x):
    return x * (1.0 / (1.0 + jnp.exp(-x)))


def _bdot(a, b):
    return jnp.dot(a.astype(jnp.bfloat16), b,
                   preferred_element_type=jnp.float32)


T_E = 2560
G_E = E // T_E


def _edge_body(hr, hc, drp, eap, w1h, w1c, w4, w1e, be1, we2, be2, wc1, bc1,
               wc2, ef_out, tdp_out):
    drb = drp[...]
    x = (_bdot(hr[...], w1h[...])
         + _bdot(hc[...], w1c[...])
         + lax.dot_general(drb, w4[...], (((0,), (0,)), ((), ())),
                           preferred_element_type=jnp.float32)
         + lax.dot_general(eap[...], w1e[...], (((0,), (0,)), ((), ())),
                           preferred_element_type=jnp.float32)
         + be1[...])
    x = _silu(x)
    ef = _silu(_bdot(x, we2[...]) + be2[...])
    cf = _silu(_bdot(ef, wc1[...]) + bc1[...])
    s = lax.dot_general(wc2[...], cf, (((1,), (1,)), ((), ())),
                        preferred_element_type=jnp.float32)
    tr = jnp.clip(drb[:3, :] * s, -100.0, 100.0)
    ef_out[...] = ef
    tdp_out[:3, :] = tr
    tdp_out[3:4, :] = jnp.ones((1, T_E), jnp.float32)


def _edge_mlp(hr, hc, drp, eap, w1h, w1c, w4, w1e, be1, we2, be2, wc1, bc1,
              wc2):
    full = lambda shape: pl.BlockSpec(shape, lambda i: (0, 0))
    return pl.pallas_call(
        _edge_body,
        grid=(G_E,),
        in_specs=[
            pl.BlockSpec((T_E, D), lambda i: (i, 0)),
            pl.BlockSpec((T_E, D), lambda i: (i, 0)),
            pl.BlockSpec((4, T_E), lambda i: (0, i)),
            pl.BlockSpec((D_EDGE, T_E), lambda i: (0, i)),
            full((D, HID)),
            full((D, HID)),
            full((4, HID)),
            full((D_EDGE, HID)),
            full((1, HID)),
            full((HID, HID)),
            full((1, HID)),
            full((HID, HID)),
            full((1, HID)),
            full((1, HID)),
        ],
        out_specs=[
            pl.BlockSpec((T_E, D), lambda i: (i, 0)),
            pl.BlockSpec((4, T_E), lambda i: (0, i)),
        ],
        out_shape=[
            jax.ShapeDtypeStruct((E, D), jnp.float32),
            jax.ShapeDtypeStruct((4, E), jnp.float32),
        ],
        compiler_params=pltpu.CompilerParams(
            dimension_semantics=("arbitrary",)),
    )(hr, hc, drp, eap, w1h, w1c, w4, w1e, be1, we2, be2, wc1, bc1, wc2)


T_N = 2000
G_N = N // T_N


def _node_body(h, coordp, acch, acc4p, wn1h, wn1a, bn1, wn2, bn2, h_out,
               coordp_out):
    aggh = acch[0] + acch[1]
    y = _silu(_bdot(h[...], wn1h[...]) + _bdot(aggh, wn1a[...]) + bn1[...])
    h_out[...] = h[...] + _bdot(y, wn2[...]) + bn2[...]

    @pl.when(pl.program_id(0) == 0)
    def _():
        a4 = acc4p[0]
        for k in range(1, NW):
            a4 = a4 + acc4p[k]
        num = a4[:3, :]
        cnt = a4[3:4, :]
        coordp_out[...] = coordp[...] + num / jnp.maximum(cnt, 1.0)


def _node_mlp(h, coordp, acch, acc4p, wn1h, wn1a, bn1, wn2, bn2):
    full = lambda shape: pl.BlockSpec(shape, lambda i: (0, 0))
    return pl.pallas_call(
        _node_body,
        grid=(G_N,),
        in_specs=[
            pl.BlockSpec((T_N, D), lambda i: (i, 0)),
            pl.BlockSpec((3, N), lambda i: (0, 0)),
            pl.BlockSpec((NC, T_N, D), lambda i: (0, i, 0)),
            pl.BlockSpec((NW, 4, N), lambda i: (0, 0, 0)),
            full((D, HID)),
            full((D, HID)),
            full((1, HID)),
            full((HID, D)),
            full((1, D)),
        ],
        out_specs=[
            pl.BlockSpec((T_N, D), lambda i: (i, 0)),
            pl.BlockSpec((3, N), lambda i: (0, 0)),
        ],
        out_shape=[
            jax.ShapeDtypeStruct((N, D), jnp.float32),
            jax.ShapeDtypeStruct((3, N), jnp.float32),
        ],
        compiler_params=pltpu.CompilerParams(
            dimension_semantics=("arbitrary",)),
    )(h, coordp, acch, acc4p, wn1h, wn1a, bn1, wn2, bn2)


def kernel(h, coord, edge_attr, W_e1, b_e1, W_e2, b_e2, W_n1, b_n1, W_n2,
           b_n2, W_c1, b_c1, W_c2, edge_index):
    bf = jnp.bfloat16
    row = edge_index[0]
    col = edge_index[1]
    coordf = coord.reshape(-1)

    hr, hc, drp = _gather(h, coordf, row, col)

    w4 = jnp.concatenate(
        [jnp.zeros((3, HID), jnp.float32), W_e1[2 * D:2 * D + 1]], axis=0)

    ef, tdp = _edge_mlp(
        hr, hc, drp, edge_attr.T,
        W_e1[:D].astype(bf), W_e1[D:2 * D].astype(bf), w4,
        W_e1[2 * D + 1:], b_e1.reshape(1, HID),
        W_e2.astype(bf), b_e2.reshape(1, HID),
        W_c1.astype(bf), b_c1.reshape(1, HID), W_c2.reshape(1, HID))

    acch, acc4f = _scatter(ef, tdp, row,
                           jnp.zeros((QK, D), jnp.float32),
                           jnp.zeros((4 * N,), jnp.float32))
    acc4p = acc4f.reshape(NW, 4, N)

    h_out, coordp_out = _node_mlp(
        h, coord.T, acch, acc4p,
        W_n1[:D].astype(bf), W_n1[D:].astype(bf), b_n1.reshape(1, HID),
        W_n2.astype(bf), b_n2.reshape(1, D))
    return (h_out, coordp_out.T)


# speedup vs baseline: 6.0576x; 1.2664x over previous
import functools

import jax
import jax.numpy as jnp
from jax import lax
from jax.experimental import pallas as pl
from jax.experimental.pallas import tpu as pltpu
from jax.experimental.pallas import tpu_sc as plsc

N = 10000
E = 320000
D = 128
D_EDGE = 4
HID = 128

NC = 2
NS = 16
NW = NC * NS
CHK = 128
QK = 16
NCHT = E // CHK
NFULL = NCHT // NW
NTAIL = NCHT - NFULL * NW
L = 16

NACC = 10240
RPT = NACC // NS

_mesh = plsc.VectorSubcoreMesh(core_axis_name="c", subcore_axis_name="s")


@functools.partial(
    pl.kernel,
    mesh=_mesh,
    compiler_params=pltpu.CompilerParams(needs_layout_passes=False),
    out_type=[
        jax.ShapeDtypeStruct((E, D), jnp.float32),
        jax.ShapeDtypeStruct((E, D), jnp.float32),
        jax.ShapeDtypeStruct((4, E), jnp.float32),
    ],
    scratch_types=[
        pltpu.VMEM((CHK,), jnp.int32),
        pltpu.VMEM((CHK,), jnp.int32),
        pltpu.VMEM((CHK, D), jnp.float32),
        pltpu.VMEM((CHK, D), jnp.float32),
        pltpu.VMEM((4, CHK), jnp.float32),
        pltpu.VMEM((3 * N,), jnp.float32),
        pltpu.SemaphoreType.DMA,
        pltpu.SemaphoreType.DMA,
    ],
)
def _gather(h_hbm, coordf_hbm, row_hbm, col_hbm, outr, outc, outd,
            idx_r, idx_c, rows_r, rows_c, dr_v, coord_v, sem1, sem2):
    wid = lax.axis_index("s") * NC + lax.axis_index("c")
    pltpu.sync_copy(coordf_hbm, coord_v)

    def chunk(ck):
        base = ck * CHK
        a1 = pltpu.async_copy(row_hbm.at[pl.ds(base, CHK)], idx_r, sem1)
        a2 = pltpu.async_copy(col_hbm.at[pl.ds(base, CHK)], idx_c, sem2)
        a1.wait()
        g1 = pltpu.async_copy(h_hbm.at[idx_r], rows_r, sem1)
        a2.wait()
        g2 = pltpu.async_copy(h_hbm.at[idx_c], rows_c, sem2)
        for g in range(CHK // L):
            ri3 = idx_r[pl.ds(g * L, L)] * 3
            ci3 = idx_c[pl.ds(g * L, L)] * 3
            dx = (plsc.load_gather(coord_v, [ri3])
                  - plsc.load_gather(coord_v, [ci3]))
            dy = (plsc.load_gather(coord_v, [ri3 + 1])
                  - plsc.load_gather(coord_v, [ci3 + 1]))
            dz = (plsc.load_gather(coord_v, [ri3 + 2])
                  - plsc.load_gather(coord_v, [ci3 + 2]))
            dr_v[0, pl.ds(g * L, L)] = dx
            dr_v[1, pl.ds(g * L, L)] = dy
            dr_v[2, pl.ds(g * L, L)] = dz
            dr_v[3, pl.ds(g * L, L)] = dx * dx + dy * dy + dz * dz
        pltpu.sync_copy(dr_v, outd.at[:, pl.ds(base, CHK)])
        g1.wait()
        o1 = pltpu.async_copy(rows_r, outr.at[pl.ds(base, CHK)], sem1)
        g2.wait()
        o2 = pltpu.async_copy(rows_c, outc.at[pl.ds(base, CHK)], sem2)
        o1.wait()
        o2.wait()

    def body(j, carry):
        chunk(j * NW + wid)
        return carry

    lax.fori_loop(0, NFULL, body, 0)

    @pl.when(wid < NTAIL)
    def _():
        chunk(NFULL * NW + wid)


@functools.partial(
    pl.kernel,
    mesh=_mesh,
    compiler_params=pltpu.CompilerParams(needs_layout_passes=False),
    out_type=[
        jax.ShapeDtypeStruct((NC, NACC, D), jnp.float32),
        jax.ShapeDtypeStruct((NW, 4 * N), jnp.float32),
    ],
    scratch_types=[
        pltpu.VMEM((CHK // QK, QK), jnp.int32),
        pltpu.VMEM((QK, D), jnp.float32),
        pltpu.VMEM((QK, D), jnp.float32),
        pltpu.VMEM((4, CHK), jnp.float32),
        pltpu.VMEM((4 * N,), jnp.float32),
        pltpu.VMEM_SHARED((NACC, D), jnp.float32),
        pltpu.SemaphoreType.DMA,
        pltpu.SemaphoreType.DMA,
        pltpu.SemaphoreType.DMA,
        pltpu.SemaphoreType.DMA,
    ],
)
def _scatter(ef_hbm, tdp_hbm, row2_hbm, z2_hbm, zf_hbm, outh, out4,
             idx2, data0, data1, td_v, acc4_v, acch_sh,
             sem_i, sem_t, sem_d0, sem_d1):
    cid = lax.axis_index("c")
    sid = lax.axis_index("s")
    wid = sid * NC + cid

    pltpu.sync_copy(z2_hbm, acch_sh.at[pl.ds(sid * RPT, RPT)])
    pltpu.sync_copy(zf_hbm, acc4_v)
    plsc.subcore_barrier()

    def reg_scatter(klo, khi):
        for k in range(klo, khi):
            ids = idx2[k, pl.ds(0, L)]
            for p in range(4):
                val = td_v[p, pl.ds(k * L, L)]
                plsc.addupdate_scatter(acc4_v, [ids + p * N], val)

    def chunk(ck):
        base = ck * CHK
        rbase = ck * (CHK // QK)
        a_i = pltpu.async_copy(row2_hbm.at[pl.ds(rbase, CHK // QK)], idx2,
                               sem_i)
        a_t = pltpu.async_copy(tdp_hbm.at[:, pl.ds(base, CHK)], td_v, sem_t)
        bufs = (data0, data1)
        sems = (sem_d0, sem_d1)
        pend = [
            pltpu.async_copy(ef_hbm.at[pl.ds(base, QK)], data0, sem_d0),
            pltpu.async_copy(ef_hbm.at[pl.ds(base + QK, QK)], data1, sem_d1),
        ]
        a_i.wait()
        a_t.wait()
        nq = CHK // QK
        for q in range(nq):
            pend[q].wait()
            pltpu.sync_copy(bufs[q % 2], acch_sh.at[idx2.at[q]], add=True)
            if q + 2 < nq:
                pend.append(pltpu.async_copy(
                    ef_hbm.at[pl.ds(base + (q + 2) * QK, QK)],
                    bufs[q % 2], sems[q % 2]))
            reg_scatter(q, q + 1)

    def body(j, carry):
        chunk(j * NW + wid)
        return carry

    lax.fori_loop(0, NFULL, body, 0)

    @pl.when(wid < NTAIL)
    def _():
        chunk(NFULL * NW + wid)

    pltpu.sync_copy(acc4_v, out4.at[wid])
    plsc.subcore_barrier()

    off = sid * RPT
    pltpu.sync_copy(acch_sh.at[pl.ds(off, RPT)],
                    outh.at[cid, pl.ds(off, RPT)])


def _silu(x):
    return x * (1.0 / (1.0 + jnp.exp(-x)))


def _bdot(a, b):
    return jnp.dot(a.astype(jnp.bfloat16), b,
                   preferred_element_type=jnp.float32)


T_E = 2560
G_E = E // T_E


def _edge_body(hr, hc, drp, eap, w1h, w1c, w4, w1e, be1, we2, be2, wc1, bc1,
               wc2, ef_out, tdp_out):
    drb = drp[...]
    x = (_bdot(hr[...], w1h[...])
         + _bdot(hc[...], w1c[...])
         + lax.dot_general(drb, w4[...], (((0,), (0,)), ((), ())),
                           preferred_element_type=jnp.float32)
         + lax.dot_general(eap[...], w1e[...], (((0,), (0,)), ((), ())),
                           preferred_element_type=jnp.float32)
         + be1[...])
    x = _silu(x)
    ef = _silu(_bdot(x, we2[...]) + be2[...])
    cf = _silu(_bdot(ef, wc1[...]) + bc1[...])
    s = lax.dot_general(wc2[...], cf, (((1,), (1,)), ((), ())),
                        preferred_element_type=jnp.float32)
    tr = jnp.clip(drb[:3, :] * s, -100.0, 100.0)
    ef_out[...] = ef
    tdp_out[:3, :] = tr
    tdp_out[3:4, :] = jnp.ones((1, T_E), jnp.float32)


def _edge_mlp(hr, hc, drp, eap, w1h, w1c, w4, w1e, be1, we2, be2, wc1, bc1,
              wc2):
    full = lambda shape: pl.BlockSpec(shape, lambda i: (0, 0))
    return pl.pallas_call(
        _edge_body,
        grid=(G_E,),
        in_specs=[
            pl.BlockSpec((T_E, D), lambda i: (i, 0)),
            pl.BlockSpec((T_E, D), lambda i: (i, 0)),
            pl.BlockSpec((4, T_E), lambda i: (0, i)),
            pl.BlockSpec((D_EDGE, T_E), lambda i: (0, i)),
            full((D, HID)),
            full((D, HID)),
            full((4, HID)),
            full((D_EDGE, HID)),
            full((1, HID)),
            full((HID, HID)),
            full((1, HID)),
            full((HID, HID)),
            full((1, HID)),
            full((1, HID)),
        ],
        out_specs=[
            pl.BlockSpec((T_E, D), lambda i: (i, 0)),
            pl.BlockSpec((4, T_E), lambda i: (0, i)),
        ],
        out_shape=[
            jax.ShapeDtypeStruct((E, D), jnp.float32),
            jax.ShapeDtypeStruct((4, E), jnp.float32),
        ],
        compiler_params=pltpu.CompilerParams(
            dimension_semantics=("arbitrary",)),
    )(hr, hc, drp, eap, w1h, w1c, w4, w1e, be1, we2, be2, wc1, bc1, wc2)


T_N = 2000
G_N = N // T_N


def _node_body(h, coordp, acch, acc4p, wn1h, wn1a, bn1, wn2, bn2, h_out,
               coordp_out):
    aggh = acch[0] + acch[1]
    y = _silu(_bdot(h[...], wn1h[...]) + _bdot(aggh, wn1a[...]) + bn1[...])
    h_out[...] = h[...] + _bdot(y, wn2[...]) + bn2[...]

    @pl.when(pl.program_id(0) == 0)
    def _():
        a4 = acc4p[0]
        for k in range(1, NW):
            a4 = a4 + acc4p[k]
        num = a4[:3, :]
        cnt = a4[3:4, :]
        coordp_out[...] = coordp[...] + num / jnp.maximum(cnt, 1.0)


def _node_mlp(h, coordp, acch, acc4p, wn1h, wn1a, bn1, wn2, bn2):
    full = lambda shape: pl.BlockSpec(shape, lambda i: (0, 0))
    return pl.pallas_call(
        _node_body,
        grid=(G_N,),
        in_specs=[
            pl.BlockSpec((T_N, D), lambda i: (i, 0)),
            pl.BlockSpec((3, N), lambda i: (0, 0)),
            pl.BlockSpec((NC, T_N, D), lambda i: (0, i, 0)),
            pl.BlockSpec((NW, 4, N), lambda i: (0, 0, 0)),
            full((D, HID)),
            full((D, HID)),
            full((1, HID)),
            full((HID, D)),
            full((1, D)),
        ],
        out_specs=[
            pl.BlockSpec((T_N, D), lambda i: (i, 0)),
            pl.BlockSpec((3, N), lambda i: (0, 0)),
        ],
        out_shape=[
            jax.ShapeDtypeStruct((N, D), jnp.float32),
            jax.ShapeDtypeStruct((3, N), jnp.float32),
        ],
        compiler_params=pltpu.CompilerParams(
            dimension_semantics=("arbitrary",)),
    )(h, coordp, acch, acc4p, wn1h, wn1a, bn1, wn2, bn2)


def kernel(h, coord, edge_attr, W_e1, b_e1, W_e2, b_e2, W_n1, b_n1, W_n2,
           b_n2, W_c1, b_c1, W_c2, edge_index):
    bf = jnp.bfloat16
    row = edge_index[0]
    col = edge_index[1]
    coordf = coord.reshape(-1)

    hr, hc, drp = _gather(h, coordf, row, col)

    w4 = jnp.concatenate(
        [jnp.zeros((3, HID), jnp.float32), W_e1[2 * D:2 * D + 1]], axis=0)

    ef, tdp = _edge_mlp(
        hr, hc, drp, edge_attr.T,
        W_e1[:D].astype(bf), W_e1[D:2 * D].astype(bf), w4,
        W_e1[2 * D + 1:], b_e1.reshape(1, HID),
        W_e2.astype(bf), b_e2.reshape(1, HID),
        W_c1.astype(bf), b_c1.reshape(1, HID), W_c2.reshape(1, HID))

    acch, acc4f = _scatter(ef, tdp, row.reshape(E // QK, QK),
                           jnp.zeros((RPT, D), jnp.float32),
                           jnp.zeros((4 * N,), jnp.float32))
    acc4p = acc4f.reshape(NW, 4, N)

    h_out, coordp_out = _node_mlp(
        h, coord.T, acch, acc4p,
        W_n1[:D].astype(bf), W_n1[D:].astype(bf), b_n1.reshape(1, HID),
        W_n2.astype(bf), b_n2.reshape(1, D))
    return (h_out, coordp_out.T)


# speedup vs baseline: 6.8467x; 1.1303x over previous
import functools

import jax
import jax.numpy as jnp
from jax import lax
from jax.experimental import pallas as pl
from jax.experimental.pallas import tpu as pltpu
from jax.experimental.pallas import tpu_sc as plsc

N = 10000
E = 320000
D = 128
D_EDGE = 4
HID = 128

NSEG = 2
ES = E // NSEG

NC = 2
NS = 16
NW = NC * NS
CHK = 128
QK = 16
NCHT = ES // CHK
NFULL = NCHT // NW
NTAIL = NCHT - NFULL * NW
L = 16

NACC = 10240
RPT = NACC // NS

_mesh = plsc.VectorSubcoreMesh(core_axis_name="c", subcore_axis_name="s")


@functools.partial(
    pl.kernel,
    mesh=_mesh,
    compiler_params=pltpu.CompilerParams(needs_layout_passes=False),
    out_type=[
        jax.ShapeDtypeStruct((ES, D), jnp.float32),
        jax.ShapeDtypeStruct((ES, D), jnp.float32),
        jax.ShapeDtypeStruct((4, ES), jnp.float32),
    ],
    scratch_types=[
        pltpu.VMEM((CHK,), jnp.int32),
        pltpu.VMEM((CHK,), jnp.int32),
        pltpu.VMEM((CHK, D), jnp.float32),
        pltpu.VMEM((CHK, D), jnp.float32),
        pltpu.VMEM((4, CHK), jnp.float32),
        pltpu.VMEM((3 * N,), jnp.float32),
        pltpu.SemaphoreType.DMA,
        pltpu.SemaphoreType.DMA,
    ],
)
def _gather(h_hbm, coordf_hbm, row_hbm, col_hbm, outr, outc, outd,
            idx_r, idx_c, rows_r, rows_c, dr_v, coord_v, sem1, sem2):
    wid = lax.axis_index("s") * NC + lax.axis_index("c")
    pltpu.sync_copy(coordf_hbm, coord_v)

    def chunk(ck):
        base = ck * CHK
        a1 = pltpu.async_copy(row_hbm.at[pl.ds(base, CHK)], idx_r, sem1)
        a2 = pltpu.async_copy(col_hbm.at[pl.ds(base, CHK)], idx_c, sem2)
        a1.wait()
        g1 = pltpu.async_copy(h_hbm.at[idx_r], rows_r, sem1)
        a2.wait()
        g2 = pltpu.async_copy(h_hbm.at[idx_c], rows_c, sem2)
        for g in range(CHK // L):
            ri3 = idx_r[pl.ds(g * L, L)] * 3
            ci3 = idx_c[pl.ds(g * L, L)] * 3
            dx = (plsc.load_gather(coord_v, [ri3])
                  - plsc.load_gather(coord_v, [ci3]))
            dy = (plsc.load_gather(coord_v, [ri3 + 1])
                  - plsc.load_gather(coord_v, [ci3 + 1]))
            dz = (plsc.load_gather(coord_v, [ri3 + 2])
                  - plsc.load_gather(coord_v, [ci3 + 2]))
            dr_v[0, pl.ds(g * L, L)] = dx
            dr_v[1, pl.ds(g * L, L)] = dy
            dr_v[2, pl.ds(g * L, L)] = dz
            dr_v[3, pl.ds(g * L, L)] = dx * dx + dy * dy + dz * dz
        pltpu.sync_copy(dr_v, outd.at[:, pl.ds(base, CHK)])
        g1.wait()
        o1 = pltpu.async_copy(rows_r, outr.at[pl.ds(base, CHK)], sem1)
        g2.wait()
        o2 = pltpu.async_copy(rows_c, outc.at[pl.ds(base, CHK)], sem2)
        o1.wait()
        o2.wait()

    def body(j, carry):
        chunk(j * NW + wid)
        return carry

    lax.fori_loop(0, NFULL, body, 0)

    @pl.when(wid < NTAIL)
    def _():
        chunk(NFULL * NW + wid)


@functools.partial(
    pl.kernel,
    mesh=_mesh,
    compiler_params=pltpu.CompilerParams(needs_layout_passes=False),
    out_type=[
        jax.ShapeDtypeStruct((NC, NACC, D), jnp.float32),
        jax.ShapeDtypeStruct((NW, 4 * N), jnp.float32),
    ],
    scratch_types=[
        pltpu.VMEM((CHK // QK, QK), jnp.int32),
        pltpu.VMEM((QK, D), jnp.float32),
        pltpu.VMEM((QK, D), jnp.float32),
        pltpu.VMEM((4, CHK), jnp.float32),
        pltpu.VMEM((4 * N,), jnp.float32),
        pltpu.VMEM_SHARED((NACC, D), jnp.float32),
        pltpu.SemaphoreType.DMA,
        pltpu.SemaphoreType.DMA,
        pltpu.SemaphoreType.DMA,
        pltpu.SemaphoreType.DMA,
    ],
)
def _scatter(ef_hbm, tdp_hbm, row2_hbm, z2_hbm, zf_hbm, outh, out4,
             idx2, data0, data1, td_v, acc4_v, acch_sh,
             sem_i, sem_t, sem_d0, sem_d1):
    cid = lax.axis_index("c")
    sid = lax.axis_index("s")
    wid = sid * NC + cid

    pltpu.sync_copy(z2_hbm, acch_sh.at[pl.ds(sid * RPT, RPT)])
    pltpu.sync_copy(zf_hbm, acc4_v)
    plsc.subcore_barrier()

    def reg_scatter(klo, khi):
        for k in range(klo, khi):
            ids = idx2[k, pl.ds(0, L)]
            for p in range(4):
                val = td_v[p, pl.ds(k * L, L)]
                plsc.addupdate_scatter(acc4_v, [ids + p * N], val)

    def chunk(ck):
        base = ck * CHK
        rbase = ck * (CHK // QK)
        a_i = pltpu.async_copy(row2_hbm.at[pl.ds(rbase, CHK // QK)], idx2,
                               sem_i)
        a_t = pltpu.async_copy(tdp_hbm.at[:, pl.ds(base, CHK)], td_v, sem_t)
        bufs = (data0, data1)
        sems = (sem_d0, sem_d1)
        pend = [
            pltpu.async_copy(ef_hbm.at[pl.ds(base, QK)], data0, sem_d0),
            pltpu.async_copy(ef_hbm.at[pl.ds(base + QK, QK)], data1, sem_d1),
        ]
        a_i.wait()
        a_t.wait()
        nq = CHK // QK
        for q in range(nq):
            pend[q].wait()
            pltpu.sync_copy(bufs[q % 2], acch_sh.at[idx2.at[q]], add=True)
            if q + 2 < nq:
                pend.append(pltpu.async_copy(
                    ef_hbm.at[pl.ds(base + (q + 2) * QK, QK)],
                    bufs[q % 2], sems[q % 2]))
            reg_scatter(q, q + 1)

    def body(j, carry):
        chunk(j * NW + wid)
        return carry

    lax.fori_loop(0, NFULL, body, 0)

    @pl.when(wid < NTAIL)
    def _():
        chunk(NFULL * NW + wid)

    pltpu.sync_copy(acc4_v, out4.at[wid])
    plsc.subcore_barrier()

    off = sid * RPT
    pltpu.sync_copy(acch_sh.at[pl.ds(off, RPT)],
                    outh.at[cid, pl.ds(off, RPT)])


def _silu(x):
    return x * (1.0 / (1.0 + jnp.exp(-x)))


def _bdot(a, b):
    return jnp.dot(a.astype(jnp.bfloat16), b,
                   preferred_element_type=jnp.float32)


T_E = 1280
G_E = ES // T_E


def _edge_body(hr, hc, drp, eap, w1h, w1c, w4, w1e, be1, we2, be2, wc1, bc1,
               wc2, ef_out, tdp_out):
    drb = drp[...]
    x = (_bdot(hr[...], w1h[...])
         + _bdot(hc[...], w1c[...])
         + lax.dot_general(drb, w4[...], (((0,), (0,)), ((), ())),
                           preferred_element_type=jnp.float32)
         + lax.dot_general(eap[...], w1e[...], (((0,), (0,)), ((), ())),
                           preferred_element_type=jnp.float32)
         + be1[...])
    x = _silu(x)
    ef = _silu(_bdot(x, we2[...]) + be2[...])
    cf = _silu(_bdot(ef, wc1[...]) + bc1[...])
    s = lax.dot_general(wc2[...], cf, (((1,), (1,)), ((), ())),
                        preferred_element_type=jnp.float32)
    tr = jnp.clip(drb[:3, :] * s, -100.0, 100.0)
    ef_out[...] = ef
    tdp_out[:3, :] = tr
    tdp_out[3:4, :] = jnp.ones((1, T_E), jnp.float32)


def _edge_mlp(hr, hc, drp, eap, w1h, w1c, w4, w1e, be1, we2, be2, wc1, bc1,
              wc2):
    full = lambda shape: pl.BlockSpec(shape, lambda i: (0, 0))
    return pl.pallas_call(
        _edge_body,
        grid=(G_E,),
        in_specs=[
            pl.BlockSpec((T_E, D), lambda i: (i, 0)),
            pl.BlockSpec((T_E, D), lambda i: (i, 0)),
            pl.BlockSpec((4, T_E), lambda i: (0, i)),
            pl.BlockSpec((D_EDGE, T_E), lambda i: (0, i)),
            full((D, HID)),
            full((D, HID)),
            full((4, HID)),
            full((D_EDGE, HID)),
            full((1, HID)),
            full((HID, HID)),
            full((1, HID)),
            full((HID, HID)),
            full((1, HID)),
            full((1, HID)),
        ],
        out_specs=[
            pl.BlockSpec((T_E, D), lambda i: (i, 0)),
            pl.BlockSpec((4, T_E), lambda i: (0, i)),
        ],
        out_shape=[
            jax.ShapeDtypeStruct((ES, D), jnp.float32),
            jax.ShapeDtypeStruct((4, ES), jnp.float32),
        ],
        compiler_params=pltpu.CompilerParams(
            dimension_semantics=("arbitrary",)),
    )(hr, hc, drp, eap, w1h, w1c, w4, w1e, be1, we2, be2, wc1, bc1, wc2)


T_N = 2000
G_N = N // T_N


def _node_body(h, coordp, acch0, acch1, acc40, acc41, wn1h, wn1a, bn1, wn2,
               bn2, h_out, coordp_out):
    aggh = (acch0[0] + acch0[1]) + (acch1[0] + acch1[1])
    y = _silu(_bdot(h[...], wn1h[...]) + _bdot(aggh, wn1a[...]) + bn1[...])
    h_out[...] = h[...] + _bdot(y, wn2[...]) + bn2[...]

    @pl.when(pl.program_id(0) == 0)
    def _():
        a4 = acc40[0]
        for k in range(1, NW):
            a4 = a4 + acc40[k]
        for k in range(NW):
            a4 = a4 + acc41[k]
        num = a4[:3, :]
        cnt = a4[3:4, :]
        coordp_out[...] = coordp[...] + num / jnp.maximum(cnt, 1.0)


def _node_mlp(h, coordp, acch0, acch1, acc40, acc41, wn1h, wn1a, bn1, wn2,
              bn2):
    full = lambda shape: pl.BlockSpec(shape, lambda i: (0, 0))
    acch_spec = pl.BlockSpec((NC, T_N, D), lambda i: (0, i, 0))
    acc4_spec = pl.BlockSpec((NW, 4, N), lambda i: (0, 0, 0))
    return pl.pallas_call(
        _node_body,
        grid=(G_N,),
        in_specs=[
            pl.BlockSpec((T_N, D), lambda i: (i, 0)),
            pl.BlockSpec((3, N), lambda i: (0, 0)),
            acch_spec,
            acch_spec,
            acc4_spec,
            acc4_spec,
            full((D, HID)),
            full((D, HID)),
            full((1, HID)),
            full((HID, D)),
            full((1, D)),
        ],
        out_specs=[
            pl.BlockSpec((T_N, D), lambda i: (i, 0)),
            pl.BlockSpec((3, N), lambda i: (0, 0)),
        ],
        out_shape=[
            jax.ShapeDtypeStruct((N, D), jnp.float32),
            jax.ShapeDtypeStruct((3, N), jnp.float32),
        ],
        compiler_params=pltpu.CompilerParams(
            dimension_semantics=("arbitrary",)),
    )(h, coordp, acch0, acch1, acc40, acc41, wn1h, wn1a, bn1, wn2, bn2)


def kernel(h, coord, edge_attr, W_e1, b_e1, W_e2, b_e2, W_n1, b_n1, W_n2,
           b_n2, W_c1, b_c1, W_c2, edge_index):
    bf = jnp.bfloat16
    coordf = coord.reshape(-1)
    eap = edge_attr.T

    w4 = jnp.concatenate(
        [jnp.zeros((3, HID), jnp.float32), W_e1[2 * D:2 * D + 1]], axis=0)
    ew = (W_e1[:D].astype(bf), W_e1[D:2 * D].astype(bf), w4,
          W_e1[2 * D + 1:], b_e1.reshape(1, HID),
          W_e2.astype(bf), b_e2.reshape(1, HID),
          W_c1.astype(bf), b_c1.reshape(1, HID), W_c2.reshape(1, HID))

    z2 = jnp.zeros((RPT, D), jnp.float32)
    zf = jnp.zeros((4 * N,), jnp.float32)

    rows = [edge_index[0, s * ES:(s + 1) * ES] for s in range(NSEG)]
    cols = [edge_index[1, s * ES:(s + 1) * ES] for s in range(NSEG)]
    eaps = [eap[:, s * ES:(s + 1) * ES] for s in range(NSEG)]

    gat = [_gather(h, coordf, rows[s], cols[s]) for s in range(NSEG)]
    edg = [_edge_mlp(gat[s][0], gat[s][1], gat[s][2], eaps[s], *ew)
           for s in range(NSEG)]
    sca = [_scatter(edg[s][0], edg[s][1], rows[s].reshape(ES // QK, QK),
                    z2, zf) for s in range(NSEG)]

    h_out, coordp_out = _node_mlp(
        h, coord.T, sca[0][0], sca[1][0],
        sca[0][1].reshape(NW, 4, N), sca[1][1].reshape(NW, 4, N),
        W_n1[:D].astype(bf), W_n1[D:].astype(bf), b_n1.reshape(1, HID),
        W_n2.astype(bf), b_n2.reshape(1, D))
    return (h_out, coordp_out.T)


# speedup vs baseline: 7.3886x; 1.0791x over previous
import functools

import jax
import jax.numpy as jnp
from jax import lax
from jax.experimental import pallas as pl
from jax.experimental.pallas import tpu as pltpu
from jax.experimental.pallas import tpu_sc as plsc

N = 10000
E = 320000
D = 128
D_EDGE = 4
HID = 128

NSEG = 2
ES = E // NSEG

NC = 2
NS = 16
NW = NC * NS
CHK = 128
QK = 16
NCHT = ES // CHK
NFULL = NCHT // NW
NTAIL = NCHT - NFULL * NW
L = 16

NACC = 10240
RPT = NACC // NS

_mesh = plsc.VectorSubcoreMesh(core_axis_name="c", subcore_axis_name="s")


@functools.partial(
    pl.kernel,
    mesh=_mesh,
    compiler_params=pltpu.CompilerParams(needs_layout_passes=False),
    out_type=[
        jax.ShapeDtypeStruct((ES, D), jnp.float32),
        jax.ShapeDtypeStruct((ES, D), jnp.float32),
        jax.ShapeDtypeStruct((4, ES), jnp.float32),
    ],
    scratch_types=[
        [pltpu.VMEM((CHK,), jnp.int32) for _ in range(4)],
        [pltpu.VMEM((CHK, D), jnp.float32) for _ in range(4)],
        [pltpu.VMEM((4, CHK), jnp.float32) for _ in range(2)],
        pltpu.VMEM((3 * N,), jnp.float32),
        [pltpu.SemaphoreType.DMA for _ in range(4)],
    ],
)
def _gather(h_hbm, coordf_hbm, row_hbm, col_hbm, outr, outc, outd,
            idx, rows, dr, coord_v, sems):
    wid = lax.axis_index("s") * NC + lax.axis_index("c")
    pltpu.sync_copy(coordf_hbm, coord_v)
    tw = NFULL + jnp.where(wid < NTAIL, 1, 0)

    def slot_ck(k):
        return jnp.where(k < NFULL, k * NW + wid, NFULL * NW + wid)

    def regwork(idx_r, idx_c, dr_v, base):
        for g in range(CHK // L):
            ri3 = idx_r[pl.ds(g * L, L)] * 3
            ci3 = idx_c[pl.ds(g * L, L)] * 3
            dx = (plsc.load_gather(coord_v, [ri3])
                  - plsc.load_gather(coord_v, [ci3]))
            dy = (plsc.load_gather(coord_v, [ri3 + 1])
                  - plsc.load_gather(coord_v, [ci3 + 1]))
            dz = (plsc.load_gather(coord_v, [ri3 + 2])
                  - plsc.load_gather(coord_v, [ci3 + 2]))
            dr_v[0, pl.ds(g * L, L)] = dx
            dr_v[1, pl.ds(g * L, L)] = dy
            dr_v[2, pl.ds(g * L, L)] = dz
            dr_v[3, pl.ds(g * L, L)] = dx * dx + dy * dy + dz * dz
        pltpu.sync_copy(dr_v, outd.at[:, pl.ds(base, CHK)])

    def pair(m, carry):
        k0 = 2 * m
        k1 = 2 * m + 1
        b0 = slot_ck(k0) * CHK
        b1 = slot_ck(k1) * CHK
        has1 = k1 < tw

        a0 = pltpu.async_copy(row_hbm.at[pl.ds(b0, CHK)], idx[0], sems[0])
        a1 = pltpu.async_copy(col_hbm.at[pl.ds(b0, CHK)], idx[1], sems[1])
        a0.wait()
        g0 = pltpu.async_copy(h_hbm.at[idx[0]], rows[0], sems[0])
        a1.wait()
        g1 = pltpu.async_copy(h_hbm.at[idx[1]], rows[1], sems[1])

        @pl.when(has1)
        def _():
            a2 = pltpu.async_copy(row_hbm.at[pl.ds(b1, CHK)], idx[2],
                                  sems[2])
            a3 = pltpu.async_copy(col_hbm.at[pl.ds(b1, CHK)], idx[3],
                                  sems[3])
            a2.wait()
            pltpu.async_copy(h_hbm.at[idx[2]], rows[2], sems[2])
            a3.wait()
            pltpu.async_copy(h_hbm.at[idx[3]], rows[3], sems[3])

        regwork(idx[0], idx[1], dr[0], b0)
        g0.wait()
        o0 = pltpu.async_copy(rows[0], outr.at[pl.ds(b0, CHK)], sems[0])
        g1.wait()
        o1 = pltpu.async_copy(rows[1], outc.at[pl.ds(b0, CHK)], sems[1])

        @pl.when(has1)
        def _():
            regwork(idx[2], idx[3], dr[1], b1)
            pltpu.make_async_copy(h_hbm.at[idx[2]], rows[2], sems[2]).wait()
            pltpu.async_copy(rows[2], outr.at[pl.ds(b1, CHK)], sems[2])
            pltpu.make_async_copy(h_hbm.at[idx[3]], rows[3], sems[3]).wait()
            pltpu.async_copy(rows[3], outc.at[pl.ds(b1, CHK)], sems[3])
            pltpu.make_async_copy(rows[2], outr.at[pl.ds(b1, CHK)],
                                  sems[2]).wait()
            pltpu.make_async_copy(rows[3], outc.at[pl.ds(b1, CHK)],
                                  sems[3]).wait()

        o0.wait()
        o1.wait()
        return carry

    lax.fori_loop(0, (NFULL + 2) // 2, pair, 0)


@functools.partial(
    pl.kernel,
    mesh=_mesh,
    compiler_params=pltpu.CompilerParams(needs_layout_passes=False),
    out_type=[
        jax.ShapeDtypeStruct((NC, NACC, D), jnp.float32),
        jax.ShapeDtypeStruct((NW, 4 * N), jnp.float32),
    ],
    scratch_types=[
        pltpu.VMEM((CHK // QK, QK), jnp.int32),
        pltpu.VMEM((QK, D), jnp.float32),
        pltpu.VMEM((QK, D), jnp.float32),
        pltpu.VMEM((4, CHK), jnp.float32),
        pltpu.VMEM((4 * N,), jnp.float32),
        pltpu.VMEM_SHARED((NACC, D), jnp.float32),
        pltpu.SemaphoreType.DMA,
        pltpu.SemaphoreType.DMA,
        pltpu.SemaphoreType.DMA,
        pltpu.SemaphoreType.DMA,
    ],
)
def _scatter(ef_hbm, tdp_hbm, row2_hbm, z2_hbm, zf_hbm, outh, out4,
             idx2, data0, data1, td_v, acc4_v, acch_sh,
             sem_i, sem_t, sem_d0, sem_d1):
    cid = lax.axis_index("c")
    sid = lax.axis_index("s")
    wid = sid * NC + cid

    pltpu.sync_copy(z2_hbm, acch_sh.at[pl.ds(sid * RPT, RPT)])
    pltpu.sync_copy(zf_hbm, acc4_v)
    plsc.subcore_barrier()

    def reg_scatter(klo, khi):
        for k in range(klo, khi):
            ids = idx2[k, pl.ds(0, L)]
            for p in range(4):
                val = td_v[p, pl.ds(k * L, L)]
                plsc.addupdate_scatter(acc4_v, [ids + p * N], val)

    def chunk(ck):
        base = ck * CHK
        rbase = ck * (CHK // QK)
        a_i = pltpu.async_copy(row2_hbm.at[pl.ds(rbase, CHK // QK)], idx2,
                               sem_i)
        a_t = pltpu.async_copy(tdp_hbm.at[:, pl.ds(base, CHK)], td_v, sem_t)
        bufs = (data0, data1)
        sems = (sem_d0, sem_d1)
        pend = [
            pltpu.async_copy(ef_hbm.at[pl.ds(base, QK)], data0, sem_d0),
            pltpu.async_copy(ef_hbm.at[pl.ds(base + QK, QK)], data1, sem_d1),
        ]
        a_i.wait()
        a_t.wait()
        nq = CHK // QK
        for q in range(nq):
            pend[q].wait()
            pltpu.sync_copy(bufs[q % 2], acch_sh.at[idx2.at[q]], add=True)
            if q + 2 < nq:
                pend.append(pltpu.async_copy(
                    ef_hbm.at[pl.ds(base + (q + 2) * QK, QK)],
                    bufs[q % 2], sems[q % 2]))
            reg_scatter(q, q + 1)

    def body(j, carry):
        chunk(j * NW + wid)
        return carry

    lax.fori_loop(0, NFULL, body, 0)

    @pl.when(wid < NTAIL)
    def _():
        chunk(NFULL * NW + wid)

    pltpu.sync_copy(acc4_v, out4.at[wid])
    plsc.subcore_barrier()

    off = sid * RPT
    pltpu.sync_copy(acch_sh.at[pl.ds(off, RPT)],
                    outh.at[cid, pl.ds(off, RPT)])


def _silu(x):
    return x * (1.0 / (1.0 + jnp.exp(-x)))


def _bdot(a, b):
    return jnp.dot(a.astype(jnp.bfloat16), b,
                   preferred_element_type=jnp.float32)


T_E = 3200
G_E = ES // T_E


def _edge_body(hr, hc, drp, eap, w1h, w1c, w4, w1e, be1, we2, be2, wc1, bc1,
               wc2, ef_out, tdp_out):
    drb = drp[...]
    x = (_bdot(hr[...], w1h[...])
         + _bdot(hc[...], w1c[...])
         + lax.dot_general(drb, w4[...], (((0,), (0,)), ((), ())),
                           preferred_element_type=jnp.float32)
         + lax.dot_general(eap[...], w1e[...], (((0,), (0,)), ((), ())),
                           preferred_element_type=jnp.float32)
         + be1[...])
    x = _silu(x)
    ef = _silu(_bdot(x, we2[...]) + be2[...])
    cf = _silu(_bdot(ef, wc1[...]) + bc1[...])
    s = lax.dot_general(wc2[...], cf, (((1,), (1,)), ((), ())),
                        preferred_element_type=jnp.float32)
    tr = jnp.clip(drb[:3, :] * s, -100.0, 100.0)
    ef_out[...] = ef
    tdp_out[:3, :] = tr
    tdp_out[3:4, :] = jnp.ones((1, T_E), jnp.float32)


def _edge_mlp(hr, hc, drp, eap, w1h, w1c, w4, w1e, be1, we2, be2, wc1, bc1,
              wc2):
    full = lambda shape: pl.BlockSpec(shape, lambda i: (0, 0))
    return pl.pallas_call(
        _edge_body,
        grid=(G_E,),
        in_specs=[
            pl.BlockSpec((T_E, D), lambda i: (i, 0)),
            pl.BlockSpec((T_E, D), lambda i: (i, 0)),
            pl.BlockSpec((4, T_E), lambda i: (0, i)),
            pl.BlockSpec((D_EDGE, T_E), lambda i: (0, i)),
            full((D, HID)),
            full((D, HID)),
            full((4, HID)),
            full((D_EDGE, HID)),
            full((1, HID)),
            full((HID, HID)),
            full((1, HID)),
            full((HID, HID)),
            full((1, HID)),
            full((1, HID)),
        ],
        out_specs=[
            pl.BlockSpec((T_E, D), lambda i: (i, 0)),
            pl.BlockSpec((4, T_E), lambda i: (0, i)),
        ],
        out_shape=[
            jax.ShapeDtypeStruct((ES, D), jnp.float32),
            jax.ShapeDtypeStruct((4, ES), jnp.float32),
        ],
        compiler_params=pltpu.CompilerParams(
            dimension_semantics=("arbitrary",)),
    )(hr, hc, drp, eap, w1h, w1c, w4, w1e, be1, we2, be2, wc1, bc1, wc2)


T_N = 2000
G_N = N // T_N


def _node_body(h, coordp, acch0, acch1, acc40, acc41, wn1h, wn1a, bn1, wn2,
               bn2, h_out, coordp_out):
    aggh = (acch0[0] + acch0[1]) + (acch1[0] + acch1[1])
    y = _silu(_bdot(h[...], wn1h[...]) + _bdot(aggh, wn1a[...]) + bn1[...])
    h_out[...] = h[...] + _bdot(y, wn2[...]) + bn2[...]

    @pl.when(pl.program_id(0) == 0)
    def _():
        a4 = acc40[0]
        for k in range(1, NW):
            a4 = a4 + acc40[k]
        for k in range(NW):
            a4 = a4 + acc41[k]
        num = a4[:3, :]
        cnt = a4[3:4, :]
        coordp_out[...] = coordp[...] + num / jnp.maximum(cnt, 1.0)


def _node_mlp(h, coordp, acch0, acch1, acc40, acc41, wn1h, wn1a, bn1, wn2,
              bn2):
    full = lambda shape: pl.BlockSpec(shape, lambda i: (0, 0))
    acch_spec = pl.BlockSpec((NC, T_N, D), lambda i: (0, i, 0))
    acc4_spec = pl.BlockSpec((NW, 4, N), lambda i: (0, 0, 0))
    return pl.pallas_call(
        _node_body,
        grid=(G_N,),
        in_specs=[
            pl.BlockSpec((T_N, D), lambda i: (i, 0)),
            pl.BlockSpec((3, N), lambda i: (0, 0)),
            acch_spec,
            acch_spec,
            acc4_spec,
            acc4_spec,
            full((D, HID)),
            full((D, HID)),
            full((1, HID)),
            full((HID, D)),
            full((1, D)),
        ],
        out_specs=[
            pl.BlockSpec((T_N, D), lambda i: (i, 0)),
            pl.BlockSpec((3, N), lambda i: (0, 0)),
        ],
        out_shape=[
            jax.ShapeDtypeStruct((N, D), jnp.float32),
            jax.ShapeDtypeStruct((3, N), jnp.float32),
        ],
        compiler_params=pltpu.CompilerParams(
            dimension_semantics=("arbitrary",)),
    )(h, coordp, acch0, acch1, acc40, acc41, wn1h, wn1a, bn1, wn2, bn2)


def kernel(h, coord, edge_attr, W_e1, b_e1, W_e2, b_e2, W_n1, b_n1, W_n2,
           b_n2, W_c1, b_c1, W_c2, edge_index):
    bf = jnp.bfloat16
    coordf = coord.reshape(-1)
    eap = edge_attr.T

    w4 = jnp.concatenate(
        [jnp.zeros((3, HID), jnp.float32), W_e1[2 * D:2 * D + 1]], axis=0)
    ew = (W_e1[:D].astype(bf), W_e1[D:2 * D].astype(bf), w4,
          W_e1[2 * D + 1:], b_e1.reshape(1, HID),
          W_e2.astype(bf), b_e2.reshape(1, HID),
          W_c1.astype(bf), b_c1.reshape(1, HID), W_c2.reshape(1, HID))

    z2 = jnp.zeros((RPT, D), jnp.float32)
    zf = jnp.zeros((4 * N,), jnp.float32)

    rows = [edge_index[0, s * ES:(s + 1) * ES] for s in range(NSEG)]
    cols = [edge_index[1, s * ES:(s + 1) * ES] for s in range(NSEG)]
    eaps = [eap[:, s * ES:(s + 1) * ES] for s in range(NSEG)]

    gat = [_gather(h, coordf, rows[s], cols[s]) for s in range(NSEG)]
    edg = [_edge_mlp(gat[s][0], gat[s][1], gat[s][2], eaps[s], *ew)
           for s in range(NSEG)]
    sca = [_scatter(edg[s][0], edg[s][1], rows[s].reshape(ES // QK, QK),
                    z2, zf) for s in range(NSEG)]

    h_out, coordp_out = _node_mlp(
        h, coord.T, sca[0][0], sca[1][0],
        sca[0][1].reshape(NW, 4, N), sca[1][1].reshape(NW, 4, N),
        W_n1[:D].astype(bf), W_n1[D:].astype(bf), b_n1.reshape(1, HID),
        W_n2.astype(bf), b_n2.reshape(1, D))
    return (h_out, coordp_out.T)


# speedup vs baseline: 8.4244x; 1.1402x over previous
import functools

import jax
import jax.numpy as jnp
from jax import lax
from jax.experimental import pallas as pl
from jax.experimental.pallas import tpu as pltpu
from jax.experimental.pallas import tpu_sc as plsc

N = 10000
E = 320000
D = 128
D_EDGE = 4
HID = 128

NSEG = 2
ES = E // NSEG

NC = 2
NS = 16
NW = NC * NS
CHK = 128
QK = 32
NCHT = ES // CHK
NFULL = NCHT // NW
NTAIL = NCHT - NFULL * NW
L = 16

NACC = N
RPT = 640
RPT15 = NACC - 15 * RPT

_mesh = plsc.VectorSubcoreMesh(core_axis_name="c", subcore_axis_name="s")


@functools.partial(
    pl.kernel,
    mesh=_mesh,
    compiler_params=pltpu.CompilerParams(needs_layout_passes=False),
    out_type=[
        jax.ShapeDtypeStruct((ES, D), jnp.float32),
        jax.ShapeDtypeStruct((ES, D), jnp.float32),
        jax.ShapeDtypeStruct((4, ES), jnp.float32),
    ],
    scratch_types=[
        [pltpu.VMEM((CHK,), jnp.int32) for _ in range(4)],
        [pltpu.VMEM((CHK, D), jnp.float32) for _ in range(4)],
        [pltpu.VMEM((4, CHK), jnp.float32) for _ in range(2)],
        pltpu.VMEM((3 * N,), jnp.float32),
        [pltpu.SemaphoreType.DMA for _ in range(4)],
    ],
)
def _gather(h_hbm, coordf_hbm, row_hbm, col_hbm, outr, outc, outd,
            idx, rows, dr, coord_v, sems):
    wid = lax.axis_index("s") * NC + lax.axis_index("c")
    pltpu.sync_copy(coordf_hbm, coord_v)
    tw = NFULL + jnp.where(wid < NTAIL, 1, 0)

    def slot_ck(k):
        return jnp.where(k < NFULL, k * NW + wid, NFULL * NW + wid)

    def regwork(idx_r, idx_c, dr_v, base):
        for g in range(CHK // L):
            ri3 = idx_r[pl.ds(g * L, L)] * 3
            ci3 = idx_c[pl.ds(g * L, L)] * 3
            dx = (plsc.load_gather(coord_v, [ri3])
                  - plsc.load_gather(coord_v, [ci3]))
            dy = (plsc.load_gather(coord_v, [ri3 + 1])
                  - plsc.load_gather(coord_v, [ci3 + 1]))
            dz = (plsc.load_gather(coord_v, [ri3 + 2])
                  - plsc.load_gather(coord_v, [ci3 + 2]))
            dr_v[0, pl.ds(g * L, L)] = dx
            dr_v[1, pl.ds(g * L, L)] = dy
            dr_v[2, pl.ds(g * L, L)] = dz
            dr_v[3, pl.ds(g * L, L)] = dx * dx + dy * dy + dz * dz
        pltpu.sync_copy(dr_v, outd.at[:, pl.ds(base, CHK)])

    def pair(m, carry):
        k0 = 2 * m
        k1 = 2 * m + 1
        b0 = slot_ck(k0) * CHK
        b1 = slot_ck(k1) * CHK
        has1 = k1 < tw

        a0 = pltpu.async_copy(row_hbm.at[pl.ds(b0, CHK)], idx[0], sems[0])
        a1 = pltpu.async_copy(col_hbm.at[pl.ds(b0, CHK)], idx[1], sems[1])
        a0.wait()
        g0 = pltpu.async_copy(h_hbm.at[idx[0]], rows[0], sems[0])
        a1.wait()
        g1 = pltpu.async_copy(h_hbm.at[idx[1]], rows[1], sems[1])

        @pl.when(has1)
        def _():
            a2 = pltpu.async_copy(row_hbm.at[pl.ds(b1, CHK)], idx[2],
                                  sems[2])
            a3 = pltpu.async_copy(col_hbm.at[pl.ds(b1, CHK)], idx[3],
                                  sems[3])
            a2.wait()
            pltpu.async_copy(h_hbm.at[idx[2]], rows[2], sems[2])
            a3.wait()
            pltpu.async_copy(h_hbm.at[idx[3]], rows[3], sems[3])

        regwork(idx[0], idx[1], dr[0], b0)
        g0.wait()
        o0 = pltpu.async_copy(rows[0], outr.at[pl.ds(b0, CHK)], sems[0])
        g1.wait()
        o1 = pltpu.async_copy(rows[1], outc.at[pl.ds(b0, CHK)], sems[1])

        @pl.when(has1)
        def _():
            regwork(idx[2], idx[3], dr[1], b1)
            pltpu.make_async_copy(h_hbm.at[idx[2]], rows[2], sems[2]).wait()
            pltpu.async_copy(rows[2], outr.at[pl.ds(b1, CHK)], sems[2])
            pltpu.make_async_copy(h_hbm.at[idx[3]], rows[3], sems[3]).wait()
            pltpu.async_copy(rows[3], outc.at[pl.ds(b1, CHK)], sems[3])
            pltpu.make_async_copy(rows[2], outr.at[pl.ds(b1, CHK)],
                                  sems[2]).wait()
            pltpu.make_async_copy(rows[3], outc.at[pl.ds(b1, CHK)],
                                  sems[3]).wait()

        o0.wait()
        o1.wait()
        return carry

    lax.fori_loop(0, (NFULL + 2) // 2, pair, 0)


@functools.partial(
    pl.kernel,
    mesh=_mesh,
    compiler_params=pltpu.CompilerParams(needs_layout_passes=False),
    out_type=[
        jax.ShapeDtypeStruct((NC, NACC, D), jnp.float32),
        jax.ShapeDtypeStruct((NW, 4 * N), jnp.float32),
    ],
    scratch_types=[
        pltpu.VMEM((CHK // QK, QK), jnp.int32),
        pltpu.VMEM((QK, D), jnp.float32),
        pltpu.VMEM((QK, D), jnp.float32),
        pltpu.VMEM((4, CHK), jnp.float32),
        pltpu.VMEM((4 * N,), jnp.float32),
        pltpu.VMEM_SHARED((NACC, D), jnp.float32),
        pltpu.SemaphoreType.DMA,
        pltpu.SemaphoreType.DMA,
        pltpu.SemaphoreType.DMA,
        pltpu.SemaphoreType.DMA,
    ],
)
def _scatter(ef_hbm, tdp_hbm, row2_hbm, z2_hbm, zf_hbm, outh, out4,
             idx2, data0, data1, td_v, acc4_v, acch_sh,
             sem_i, sem_t, sem_d0, sem_d1):
    cid = lax.axis_index("c")
    sid = lax.axis_index("s")
    wid = sid * NC + cid

    @pl.when(sid < NS - 1)
    def _():
        pltpu.sync_copy(z2_hbm, acch_sh.at[pl.ds(sid * RPT, RPT)])

    @pl.when(sid == NS - 1)
    def _():
        pltpu.sync_copy(z2_hbm.at[pl.ds(0, RPT15)],
                        acch_sh.at[pl.ds((NS - 1) * RPT, RPT15)])

    pltpu.sync_copy(zf_hbm, acc4_v)
    plsc.subcore_barrier()

    def reg_scatter(klo, khi):
        for k in range(klo, khi):
            ids = idx2[k // 2, pl.ds((k % 2) * L, L)]
            for p in range(4):
                val = td_v[p, pl.ds(k * L, L)]
                plsc.addupdate_scatter(acc4_v, [ids + p * N], val)

    def chunk(ck):
        base = ck * CHK
        rbase = ck * (CHK // QK)
        a_i = pltpu.async_copy(row2_hbm.at[pl.ds(rbase, CHK // QK)], idx2,
                               sem_i)
        a_t = pltpu.async_copy(tdp_hbm.at[:, pl.ds(base, CHK)], td_v, sem_t)
        bufs = (data0, data1)
        sems = (sem_d0, sem_d1)
        pend = [
            pltpu.async_copy(ef_hbm.at[pl.ds(base, QK)], data0, sem_d0),
            pltpu.async_copy(ef_hbm.at[pl.ds(base + QK, QK)], data1, sem_d1),
        ]
        a_i.wait()
        a_t.wait()
        nq = CHK // QK
        for q in range(nq):
            pend[q].wait()
            pltpu.sync_copy(bufs[q % 2], acch_sh.at[idx2.at[q]], add=True)
            if q + 2 < nq:
                pend.append(pltpu.async_copy(
                    ef_hbm.at[pl.ds(base + (q + 2) * QK, QK)],
                    bufs[q % 2], sems[q % 2]))
            reg_scatter(2 * q, 2 * q + 2)

    def body(j, carry):
        chunk(j * NW + wid)
        return carry

    lax.fori_loop(0, NFULL, body, 0)

    @pl.when(wid < NTAIL)
    def _():
        chunk(NFULL * NW + wid)

    pltpu.sync_copy(acc4_v, out4.at[wid])
    plsc.subcore_barrier()

    @pl.when(sid < NS - 1)
    def _():
        off = sid * RPT
        pltpu.sync_copy(acch_sh.at[pl.ds(off, RPT)],
                        outh.at[cid, pl.ds(off, RPT)])

    @pl.when(sid == NS - 1)
    def _():
        off = (NS - 1) * RPT
        pltpu.sync_copy(acch_sh.at[pl.ds(off, RPT15)],
                        outh.at[cid, pl.ds(off, RPT15)])


def _silu(x):
    return x * (1.0 / (1.0 + jnp.exp(-x)))


def _bdot(a, b):
    return jnp.dot(a.astype(jnp.bfloat16), b,
                   preferred_element_type=jnp.float32)


T_E = 3200
G_E = ES // T_E


def _edge_body(hr, hc, drp, eap, w1h, w1c, w4, w1e, be1, we2, be2, wc1, bc1,
               wc2, ef_out, tdp_out):
    drb = drp[...]
    x = (_bdot(hr[...], w1h[...])
         + _bdot(hc[...], w1c[...])
         + lax.dot_general(drb, w4[...], (((0,), (0,)), ((), ())),
                           preferred_element_type=jnp.float32)
         + lax.dot_general(eap[...], w1e[...], (((0,), (0,)), ((), ())),
                           preferred_element_type=jnp.float32)
         + be1[...])
    x = _silu(x)
    ef = _silu(_bdot(x, we2[...]) + be2[...])
    cf = _silu(_bdot(ef, wc1[...]) + bc1[...])
    s = lax.dot_general(wc2[...], cf, (((1,), (1,)), ((), ())),
                        preferred_element_type=jnp.float32)
    tr = jnp.clip(drb[:3, :] * s, -100.0, 100.0)
    ef_out[...] = ef
    tdp_out[:3, :] = tr
    tdp_out[3:4, :] = jnp.ones((1, T_E), jnp.float32)


def _edge_mlp(hr, hc, drp, eap, w1h, w1c, w4, w1e, be1, we2, be2, wc1, bc1,
              wc2):
    full = lambda shape: pl.BlockSpec(shape, lambda i: (0, 0))
    return pl.pallas_call(
        _edge_body,
        grid=(G_E,),
        in_specs=[
            pl.BlockSpec((T_E, D), lambda i: (i, 0)),
            pl.BlockSpec((T_E, D), lambda i: (i, 0)),
            pl.BlockSpec((4, T_E), lambda i: (0, i)),
            pl.BlockSpec((D_EDGE, T_E), lambda i: (0, i)),
            full((D, HID)),
            full((D, HID)),
            full((4, HID)),
            full((D_EDGE, HID)),
            full((1, HID)),
            full((HID, HID)),
            full((1, HID)),
            full((HID, HID)),
            full((1, HID)),
            full((1, HID)),
        ],
        out_specs=[
            pl.BlockSpec((T_E, D), lambda i: (i, 0)),
            pl.BlockSpec((4, T_E), lambda i: (0, i)),
        ],
        out_shape=[
            jax.ShapeDtypeStruct((ES, D), jnp.float32),
            jax.ShapeDtypeStruct((4, ES), jnp.float32),
        ],
        compiler_params=pltpu.CompilerParams(
            dimension_semantics=("arbitrary",)),
    )(hr, hc, drp, eap, w1h, w1c, w4, w1e, be1, we2, be2, wc1, bc1, wc2)


T_N = 2000
G_N = N // T_N


def _node_body(h, coordp, acch0, acch1, acc40, acc41, wn1h, wn1a, bn1, wn2,
               bn2, h_out, coordp_out):
    aggh = (acch0[0] + acch0[1]) + (acch1[0] + acch1[1])
    y = _silu(_bdot(h[...], wn1h[...]) + _bdot(aggh, wn1a[...]) + bn1[...])
    h_out[...] = h[...] + _bdot(y, wn2[...]) + bn2[...]

    @pl.when(pl.program_id(0) == 0)
    def _():
        a4 = acc40[0]
        for k in range(1, NW):
            a4 = a4 + acc40[k]
        for k in range(NW):
            a4 = a4 + acc41[k]
        num = a4[:3, :]
        cnt = a4[3:4, :]
        coordp_out[...] = coordp[...] + num / jnp.maximum(cnt, 1.0)


def _node_mlp(h, coordp, acch0, acch1, acc40, acc41, wn1h, wn1a, bn1, wn2,
              bn2):
    full = lambda shape: pl.BlockSpec(shape, lambda i: (0, 0))
    acch_spec = pl.BlockSpec((NC, T_N, D), lambda i: (0, i, 0))
    acc4_spec = pl.BlockSpec((NW, 4, N), lambda i: (0, 0, 0))
    return pl.pallas_call(
        _node_body,
        grid=(G_N,),
        in_specs=[
            pl.BlockSpec((T_N, D), lambda i: (i, 0)),
            pl.BlockSpec((3, N), lambda i: (0, 0)),
            acch_spec,
            acch_spec,
            acc4_spec,
            acc4_spec,
            full((D, HID)),
            full((D, HID)),
            full((1, HID)),
            full((HID, D)),
            full((1, D)),
        ],
        out_specs=[
            pl.BlockSpec((T_N, D), lambda i: (i, 0)),
            pl.BlockSpec((3, N), lambda i: (0, 0)),
        ],
        out_shape=[
            jax.ShapeDtypeStruct((N, D), jnp.float32),
            jax.ShapeDtypeStruct((3, N), jnp.float32),
        ],
        compiler_params=pltpu.CompilerParams(
            dimension_semantics=("arbitrary",)),
    )(h, coordp, acch0, acch1, acc40, acc41, wn1h, wn1a, bn1, wn2, bn2)


def kernel(h, coord, edge_attr, W_e1, b_e1, W_e2, b_e2, W_n1, b_n1, W_n2,
           b_n2, W_c1, b_c1, W_c2, edge_index):
    bf = jnp.bfloat16
    coordf = coord.reshape(-1)
    eap = edge_attr.T

    w4 = jnp.concatenate(
        [jnp.zeros((3, HID), jnp.float32), W_e1[2 * D:2 * D + 1]], axis=0)
    ew = (W_e1[:D].astype(bf), W_e1[D:2 * D].astype(bf), w4,
          W_e1[2 * D + 1:], b_e1.reshape(1, HID),
          W_e2.astype(bf), b_e2.reshape(1, HID),
          W_c1.astype(bf), b_c1.reshape(1, HID), W_c2.reshape(1, HID))

    z2 = jnp.zeros((RPT, D), jnp.float32)
    zf = jnp.zeros((4 * N,), jnp.float32)

    rows = [edge_index[0, s * ES:(s + 1) * ES] for s in range(NSEG)]
    cols = [edge_index[1, s * ES:(s + 1) * ES] for s in range(NSEG)]
    eaps = [eap[:, s * ES:(s + 1) * ES] for s in range(NSEG)]

    gat = [_gather(h, coordf, rows[s], cols[s]) for s in range(NSEG)]
    edg = [_edge_mlp(gat[s][0], gat[s][1], gat[s][2], eaps[s], *ew)
           for s in range(NSEG)]
    sca = [_scatter(edg[s][0], edg[s][1], rows[s].reshape(ES // QK, QK),
                    z2, zf) for s in range(NSEG)]

    h_out, coordp_out = _node_mlp(
        h, coord.T, sca[0][0], sca[1][0],
        sca[0][1].reshape(NW, 4, N), sca[1][1].reshape(NW, 4, N),
        W_n1[:D].astype(bf), W_n1[D:].astype(bf), b_n1.reshape(1, HID),
        W_n2.astype(bf), b_n2.reshape(1, D))
    return (h_out, coordp_out.T)
